# baseline (speedup 1.0000x reference)
import functools

import jax
import jax.numpy as jnp
from jax.experimental import pallas as pl
from jax.experimental.pallas import tpu as pltpu


def _vq_body(z_ref, emb_ref, idx_ref, q_ref, loss_ref):
    zbt = z_ref[0]
    emb = emb_ref[...]
    e2h = 0.5 * jnp.sum(emb * emb, axis=1)
    mmt = jax.lax.dot_general(emb, zbt, (((1,), (0,)), ((), ())),
                              preferred_element_type=jnp.float32)
    dt = e2h[:, None] - mmt
    idx = jnp.argmin(dt, axis=0).astype(jnp.int32)
    dmin = jnp.min(dt, axis=0)
    z2 = jnp.sum(zbt * zbt, axis=0)
    onehot = (jax.lax.broadcasted_iota(jnp.int32, dt.shape, 0)
              == idx[None, :]).astype(jnp.float32)
    qt = jax.lax.dot_general(emb, onehot, (((0,), (0,)), ((), ())),
                             preferred_element_type=jnp.float32)
    idx_ref[0, 0, :] = idx
    q_ref[0] = qt
    loss_ref[...] = jnp.sum(2.0 * dmin + z2).reshape(1, 1, 1)


def kernel(z_e, embedding):
    batch, ch, w, h = z_e.shape
    n_codes = embedding.shape[0]
    pix = w * h
    z3 = z_e.reshape(batch, ch, pix)

    idx3, q3, loss_parts = pl.pallas_call(
        _vq_body,
        grid=(batch,),
        in_specs=[
            pl.BlockSpec((1, ch, pix), lambda i: (i, 0, 0)),
            pl.BlockSpec((n_codes, ch), lambda i: (0, 0)),
        ],
        out_specs=[
            pl.BlockSpec((1, 1, pix), lambda i: (i, 0, 0)),
            pl.BlockSpec((1, ch, pix), lambda i: (i, 0, 0)),
            pl.BlockSpec((1, 1, 1), lambda i: (i, 0, 0)),
        ],
        out_shape=[
            jax.ShapeDtypeStruct((batch, 1, pix), jnp.int32),
            jax.ShapeDtypeStruct((batch, ch, pix), jnp.float32),
            jax.ShapeDtypeStruct((batch, 1, 1), jnp.float32),
        ],
        compiler_params=pltpu.CompilerParams(
            dimension_semantics=("arbitrary",)),
    )(z3, embedding)

    indices = idx3.reshape(batch * pix)
    quantized_out = q3.reshape(batch, ch, w, h)
    vq_loss = jnp.sum(loss_parts) * (1.25 / (batch * pix * ch))
    return quantized_out, indices, vq_loss


# speedup vs baseline: 1.0529x; 1.0529x over previous
import functools

import jax
import jax.numpy as jnp
from jax.experimental import pallas as pl
from jax.experimental.pallas import tpu as pltpu

_BPB = 2


def _vq_body(z_ref, emb_ref, idx_ref, q_ref, loss_ref):
    emb = emb_ref[...]
    zbt = jnp.concatenate([z_ref[b] for b in range(_BPB)], axis=1)
    e2h = 0.5 * jnp.sum(emb * emb, axis=1)
    mmt = jax.lax.dot_general(emb, zbt, (((1,), (0,)), ((), ())),
                              preferred_element_type=jnp.float32)
    dt = e2h[:, None] - mmt
    idx = jnp.argmin(dt, axis=0).astype(jnp.int32)
    dmin = jnp.min(dt, axis=0)
    z2 = jnp.sum(zbt * zbt, axis=0)
    onehot = (jax.lax.broadcasted_iota(jnp.int32, dt.shape, 0)
              == idx[None, :]).astype(jnp.float32)
    qt = jax.lax.dot_general(emb, onehot, (((0,), (0,)), ((), ())),
                             preferred_element_type=jnp.float32)
    idx_ref[0, 0, :] = idx
    pix = qt.shape[1] // _BPB
    for b in range(_BPB):
        q_ref[b] = qt[:, b * pix:(b + 1) * pix]
    loss_ref[...] = jnp.sum(2.0 * dmin + z2).reshape(1, 1, 1)


def kernel(z_e, embedding):
    batch, ch, w, h = z_e.shape
    n_codes = embedding.shape[0]
    pix = w * h
    nb = batch // _BPB
    z3 = z_e.reshape(batch, ch, pix)

    idx3, q3, loss_parts = pl.pallas_call(
        _vq_body,
        grid=(nb,),
        in_specs=[
            pl.BlockSpec((_BPB, ch, pix), lambda i: (i, 0, 0)),
            pl.BlockSpec((n_codes, ch), lambda i: (0, 0)),
        ],
        out_specs=[
            pl.BlockSpec((1, 1, _BPB * pix), lambda i: (i, 0, 0)),
            pl.BlockSpec((_BPB, ch, pix), lambda i: (i, 0, 0)),
            pl.BlockSpec((1, 1, 1), lambda i: (i, 0, 0)),
        ],
        out_shape=[
            jax.ShapeDtypeStruct((nb, 1, _BPB * pix), jnp.int32),
            jax.ShapeDtypeStruct((batch, ch, pix), jnp.float32),
            jax.ShapeDtypeStruct((nb, 1, 1), jnp.float32),
        ],
        compiler_params=pltpu.CompilerParams(
            dimension_semantics=("arbitrary",)),
    )(z3, embedding)

    indices = idx3.reshape(batch * pix)
    quantized_out = q3.reshape(batch, ch, w, h)
    vq_loss = jnp.sum(loss_parts) * (1.25 / (batch * pix * ch))
    return quantized_out, indices, vq_loss


# speedup vs baseline: 1.0624x; 1.0090x over previous
import functools

import jax
import jax.numpy as jnp
from jax.experimental import pallas as pl
from jax.experimental.pallas import tpu as pltpu

_BPB = 4


def _vq_body(z_ref, emb_ref, idx_ref, q_ref, loss_ref):
    emb = emb_ref[...]
    zbt = jnp.concatenate([z_ref[b] for b in range(_BPB)], axis=1)
    e2h = 0.5 * jnp.sum(emb * emb, axis=1)
    mmt = jax.lax.dot_general(emb, zbt, (((1,), (0,)), ((), ())),
                              preferred_element_type=jnp.float32)
    dt = e2h[:, None] - mmt
    idx = jnp.argmin(dt, axis=0).astype(jnp.int32)
    dmin = jnp.min(dt, axis=0)
    z2 = jnp.sum(zbt * zbt, axis=0)
    onehot = (jax.lax.broadcasted_iota(jnp.int32, dt.shape, 0)
              == idx[None, :]).astype(jnp.float32)
    qt = jax.lax.dot_general(emb, onehot, (((0,), (0,)), ((), ())),
                             preferred_element_type=jnp.float32)
    idx_ref[0, 0, :] = idx
    pix = qt.shape[1] // _BPB
    for b in range(_BPB):
        q_ref[b] = qt[:, b * pix:(b + 1) * pix]
    loss_ref[...] = jnp.sum(2.0 * dmin + z2).reshape(1, 1, 1)


def kernel(z_e, embedding):
    batch, ch, w, h = z_e.shape
    n_codes = embedding.shape[0]
    pix = w * h
    nb = batch // _BPB
    z3 = z_e.reshape(batch, ch, pix)

    idx3, q3, loss_parts = pl.pallas_call(
        _vq_body,
        grid=(nb,),
        in_specs=[
            pl.BlockSpec((_BPB, ch, pix), lambda i: (i, 0, 0)),
            pl.BlockSpec((n_codes, ch), lambda i: (0, 0)),
        ],
        out_specs=[
            pl.BlockSpec((1, 1, _BPB * pix), lambda i: (i, 0, 0)),
            pl.BlockSpec((_BPB, ch, pix), lambda i: (i, 0, 0)),
            pl.BlockSpec((1, 1, 1), lambda i: (i, 0, 0)),
        ],
        out_shape=[
            jax.ShapeDtypeStruct((nb, 1, _BPB * pix), jnp.int32),
            jax.ShapeDtypeStruct((batch, ch, pix), jnp.float32),
            jax.ShapeDtypeStruct((nb, 1, 1), jnp.float32),
        ],
        compiler_params=pltpu.CompilerParams(
            dimension_semantics=("arbitrary",)),
    )(z3, embedding)

    indices = idx3.reshape(batch * pix)
    quantized_out = q3.reshape(batch, ch, w, h)
    vq_loss = jnp.sum(loss_parts) * (1.25 / (batch * pix * ch))
    return quantized_out, indices, vq_loss


# speedup vs baseline: 1.1341x; 1.0674x over previous
import functools

import jax
import jax.numpy as jnp
from jax.experimental import pallas as pl
from jax.experimental.pallas import tpu as pltpu

_BPB = 4


def _vq_body(z_ref, emb_ref, idx_ref, q_ref, loss_ref):
    emb = emb_ref[...]
    zbt = jnp.concatenate([z_ref[b] for b in range(_BPB)], axis=1)
    e2h = 0.5 * jnp.sum(emb * emb, axis=1)
    mmt = jax.lax.dot_general(emb, zbt, (((1,), (0,)), ((), ())),
                              preferred_element_type=jnp.float32)
    dt = e2h[:, None] - mmt
    dmin = jnp.min(dt, axis=0)
    z2 = jnp.sum(zbt * zbt, axis=0)
    onehot = (dt == dmin[None, :]).astype(jnp.float32)
    jcol = jax.lax.broadcasted_iota(jnp.int32, (emb.shape[0], 1), 0
                                    ).astype(jnp.float32)
    emba = jnp.concatenate([emb, jcol], axis=1)
    qa = jax.lax.dot_general(emba, onehot, (((0,), (0,)), ((), ())),
                             preferred_element_type=jnp.float32)
    qt = qa[:-1]
    idx = qa[-1].astype(jnp.int32)
    idx_ref[0, 0, :] = idx
    pix = qt.shape[1] // _BPB
    for b in range(_BPB):
        q_ref[b] = qt[:, b * pix:(b + 1) * pix]
    loss_ref[...] = jnp.sum(2.0 * dmin + z2).reshape(1, 1, 1)


def kernel(z_e, embedding):
    batch, ch, w, h = z_e.shape
    n_codes = embedding.shape[0]
    pix = w * h
    nb = batch // _BPB
    z3 = z_e.reshape(batch, ch, pix)

    idx3, q3, loss_parts = pl.pallas_call(
        _vq_body,
        grid=(nb,),
        in_specs=[
            pl.BlockSpec((_BPB, ch, pix), lambda i: (i, 0, 0)),
            pl.BlockSpec((n_codes, ch), lambda i: (0, 0)),
        ],
        out_specs=[
            pl.BlockSpec((1, 1, _BPB * pix), lambda i: (i, 0, 0)),
            pl.BlockSpec((_BPB, ch, pix), lambda i: (i, 0, 0)),
            pl.BlockSpec((1, 1, 1), lambda i: (i, 0, 0)),
        ],
        out_shape=[
            jax.ShapeDtypeStruct((nb, 1, _BPB * pix), jnp.int32),
            jax.ShapeDtypeStruct((batch, ch, pix), jnp.float32),
            jax.ShapeDtypeStruct((nb, 1, 1), jnp.float32),
        ],
        compiler_params=pltpu.CompilerParams(
            dimension_semantics=("arbitrary",)),
    )(z3, embedding)

    indices = idx3.reshape(batch * pix)
    quantized_out = q3.reshape(batch, ch, w, h)
    vq_loss = jnp.sum(loss_parts) * (1.25 / (batch * pix * ch))
    return quantized_out, indices, vq_loss
